# baseline (speedup 1.0000x reference)
import jax
import jax.numpy as jnp
from jax.experimental import pallas as pl

_K = 1024


def _vq_body(x_ref, cb1_ref, cb2_ref, bsq_ref, idx_ref, qt_ref):
    x = x_ref[0]
    c = x.shape[0]
    ab = jax.lax.dot_general(cb1_ref[...], x, (((1,), (0,)), ((), ())),
                             preferred_element_type=jnp.float32)
    score = ab + bsq_ref[...]
    mins = jnp.min(score, axis=0, keepdims=True)
    onehot = jnp.where(score == mins, 1.0, 0.0)
    qa = jax.lax.dot_general(cb2_ref[...], onehot, (((0,), (0,)), ((), ())),
                             preferred_element_type=jnp.float32)
    idx_ref[0, 0, :] = (qa[c, :] + 0.5).astype(jnp.int32)
    qt_ref[0] = qa[:c, :]


def kernel(laten, codebook):
    b_s, c, h, w = laten.shape
    hw = h * w
    x = laten.reshape(b_s, c, hw)
    b_sq = jnp.sum(codebook * codebook, axis=1, keepdims=True)
    kio = jax.lax.iota(jnp.float32, _K)[:, None]
    cb1 = -2.0 * codebook
    cb2 = jnp.concatenate([codebook, kio], axis=1)
    idx3, qt = pl.pallas_call(
        _vq_body,
        grid=(b_s,),
        in_specs=[
            pl.BlockSpec((1, c, hw), lambda b: (b, 0, 0)),
            pl.BlockSpec((_K, c), lambda b: (0, 0)),
            pl.BlockSpec((_K, c + 1), lambda b: (0, 0)),
            pl.BlockSpec((_K, 1), lambda b: (0, 0)),
        ],
        out_specs=[
            pl.BlockSpec((1, 1, hw), lambda b: (b, 0, 0)),
            pl.BlockSpec((1, c, hw), lambda b: (b, 0, 0)),
        ],
        out_shape=[
            jax.ShapeDtypeStruct((b_s, 1, hw), jnp.int32),
            jax.ShapeDtypeStruct((b_s, c, hw), jnp.float32),
        ],
    )(x, cb1, cb2, b_sq)
    return idx3.reshape(b_s, h, w), qt.reshape(b_s, c, h, w)
